# baseline (speedup 1.0000x reference)
import functools

import jax
import jax.numpy as jnp
from jax import lax
from jax.experimental import pallas as pl
from jax.experimental.pallas import tpu as pltpu
from jax.experimental.pallas import tpu_sc as plsc

D_MODEL = 64
SCALE = 8.0
SEQ = 50
BLK = 8
BROWS = BLK * SEQ
U = 80
NGU = BROWS // U
NC = 2
NS = 16
NW = NC * NS
NGB = 4
NOB = 2
NIB = 2


def _emb_body(x_hbm, t_hbm, out_hbm, raw_v, gbuf, obuf, isem, gsem, osem):
    c = lax.axis_index("c")
    s = lax.axis_index("s")
    wid = s * NC + c
    bpw = out_hbm.shape[0] // NW
    nblocks = bpw // BLK
    nunits = nblocks * NGU
    b0 = wid * bpw
    r0 = b0 * SEQ

    def idx_copy(blk, slot):
        pltpu.async_copy(
            x_hbm.at[pl.ds(r0 + blk * BROWS, BROWS)], raw_v.at[slot],
            isem.at[slot],
        )

    def wait_idx(slot):
        pltpu.make_async_copy(
            x_hbm.at[pl.ds(0, BROWS)], raw_v.at[slot], isem.at[slot]
        ).wait()

    def fire(u):
        bslot = lax.rem(u // NGU, NIB)
        gslot = lax.rem(u, NGB)
        goff = lax.rem(u, NGU) * U
        pltpu.async_copy(
            t_hbm.at[raw_v.at[bslot, pl.ds(goff, U)]],
            gbuf.at[gslot],
            gsem.at[gslot],
        )

    def wait_gather(u):
        gslot = lax.rem(u, NGB)
        pltpu.make_async_copy(
            t_hbm.at[pl.ds(0, U)], gbuf.at[gslot], gsem.at[gslot]
        ).wait()

    def fire_out(blk):
        oslot = lax.rem(blk, NOB)
        pltpu.async_copy(
            obuf.at[oslot], out_hbm.at[pl.ds(b0 + blk * BLK, BLK)],
            osem.at[oslot],
        )

    def wait_out(oslot):
        pltpu.make_async_copy(
            obuf.at[oslot], out_hbm.at[pl.ds(0, BLK)], osem.at[oslot]
        ).wait()

    def scale_move(u):
        gslot = lax.rem(u, NGB)
        oslot = lax.rem(u // NGU, NOB)
        goff = lax.rem(u, NGU) * U

        def krow(k, _):
            r = goff + k
            i = r // SEQ
            sq = r - i * SEQ
            for cix in range(D_MODEL // 16):
                sl = pl.ds(cix * 16, 16)
                obuf[oslot, i, sq, sl] = gbuf[gslot, k, sl] * SCALE
            return 0

        lax.fori_loop(0, U, krow, 0, unroll=4)

    idx_copy(0, 0)
    wait_idx(0)
    fire(0)
    fire(1)

    def step(u, carry):
        blk = u // NGU
        pos = lax.rem(u, NGU)

        @pl.when(jnp.logical_and(pos == 0, blk + 1 < nblocks))
        def _():
            idx_copy(blk + 1, lax.rem(blk + 1, NIB))

        @pl.when(jnp.logical_and(pos == 0, blk >= NOB))
        def _():
            wait_out(lax.rem(blk, NOB))

        @pl.when(jnp.logical_and(lax.rem(u + 2, NGU) == 0,
                                 u + 2 < nunits))
        def _():
            wait_idx(lax.rem((u + 2) // NGU, NIB))

        @pl.when(u + 2 < nunits)
        def _():
            fire(u + 2)

        wait_gather(u)
        scale_move(u)

        @pl.when(pos == NGU - 1)
        def _():
            fire_out(blk)

        return carry

    lax.fori_loop(0, nunits, step, 0)

    for oslot in range(NOB):
        wait_out(oslot)


@jax.jit
def kernel(x, table):
    nb, seq = x.shape
    assert seq == SEQ and nb % (NW * BLK) == 0
    xf = x.astype(jnp.int32).reshape(-1)

    out = pl.kernel(
        _emb_body,
        out_type=jax.ShapeDtypeStruct((nb, SEQ, D_MODEL), jnp.float32),
        mesh=plsc.VectorSubcoreMesh(core_axis_name="c", subcore_axis_name="s"),
        scratch_types=[
            pltpu.VMEM((NIB, BROWS), jnp.int32),
            pltpu.VMEM((NGB, U, D_MODEL), jnp.float32),
            pltpu.VMEM((NOB, BLK, SEQ, D_MODEL), jnp.float32),
            pltpu.SemaphoreType.DMA((NIB,)),
            pltpu.SemaphoreType.DMA((NGB,)),
            pltpu.SemaphoreType.DMA((NOB,)),
        ],
        compiler_params=pltpu.CompilerParams(use_tc_tiling_on_sc=False),
    )(xf, table)
    return out


# speedup vs baseline: 1.2671x; 1.2671x over previous
import jax
import jax.numpy as jnp
from jax import lax
from jax.experimental import pallas as pl
from jax.experimental.pallas import tpu as pltpu
from jax.experimental.pallas import tpu_sc as plsc

D_MODEL = 64
SCALE = 8.0
GRP = 128
K = 2
NBUF = 4
NC = 2
NS = 16
NW = NC * NS


def _emb_body(x_hbm, t_hbm, out_hbm, idx_v, bufs, gsem, osem):
    c = lax.axis_index("c")
    s = lax.axis_index("s")
    wid = s * NC + c
    gpw = x_hbm.shape[0] // NW
    nchunks = gpw // K
    grp_base = wid * gpw

    pltpu.sync_copy(x_hbm.at[pl.ds(grp_base, gpw)], idx_v)

    def fire(ch, b):
        for j in range(K):
            pltpu.async_copy(
                t_hbm.at[idx_v.at[ch * K + j]],
                bufs.at[b, j],
                gsem.at[b],
            )

    def wait_gather(b):
        pltpu.make_async_copy(
            out_hbm.at[pl.ds(0, K)], bufs.at[b], gsem.at[b]
        ).wait()

    def fire_out(ch, b):
        pltpu.async_copy(
            bufs.at[b], out_hbm.at[pl.ds(grp_base + ch * K, K)],
            osem.at[b],
        )

    def wait_out(b):
        pltpu.make_async_copy(
            bufs.at[b], out_hbm.at[pl.ds(0, K)], osem.at[b]
        ).wait()

    def scale(b):
        def scale_row(r, _):
            for j in range(K):
                for cix in range(D_MODEL // 16):
                    sl = pl.ds(cix * 16, 16)
                    bufs[b, j, r, sl] = bufs[b, j, r, sl] * SCALE
            return 0

        lax.fori_loop(0, GRP, scale_row, 0, unroll=4)

    for b in range(NBUF - 1):
        fire(b, b)

    def outer(i, carry):
        for b in range(NBUF):
            ch = i * NBUF + b
            nb = (b + NBUF - 1) % NBUF
            nch = ch + NBUF - 1

            @pl.when(jnp.logical_and(nch < nchunks, nch >= NBUF))
            def _():
                wait_out(nb)
                fire(nch, nb)

            @pl.when(jnp.logical_and(nch < nchunks, nch < NBUF))
            def _():
                fire(nch, nb)

            wait_gather(b)
            scale(b)
            fire_out(ch, b)
        return carry

    lax.fori_loop(0, nchunks // NBUF, outer, 0)

    for b in range(NBUF):
        wait_out(b)


@jax.jit
def kernel(x, table):
    orig_shape = x.shape
    b = x.size
    assert b % (NW * GRP * K * NBUF) == 0
    ngroups = b // GRP
    xi = x.reshape(ngroups, GRP).astype(jnp.int32)

    out = pl.kernel(
        _emb_body,
        out_type=jax.ShapeDtypeStruct((ngroups, GRP, D_MODEL), jnp.float32),
        mesh=plsc.VectorSubcoreMesh(core_axis_name="c", subcore_axis_name="s"),
        scratch_types=[
            pltpu.VMEM((ngroups // NW, GRP), jnp.int32),
            pltpu.VMEM((NBUF, K, GRP, D_MODEL), jnp.float32),
            pltpu.SemaphoreType.DMA((NBUF,)),
            pltpu.SemaphoreType.DMA((NBUF,)),
        ],
        compiler_params=pltpu.CompilerParams(use_tc_tiling_on_sc=False),
    )(xi, table)
    return out.reshape(*orig_shape, D_MODEL)
